# baseline (speedup 1.0000x reference)
import jax
import jax.numpy as jnp
from jax import lax
from jax.experimental import pallas as pl
from jax.experimental.pallas import tpu as pltpu
from jax.experimental.pallas import tpu_sc as plsc

N = 10000
E = 320000
D = 128
H1, H2, H3 = 128, 64, 32

NC, NS, L = 2, 16, 16
NW = NC * NS
CH = 80
EW = E // NW
NCHW = EW // CH
NPAD = 10240
DUMMY = N
SHIFT = 14
MASK = (1 << SHIFT) - 1
ZR = NPAD // NS

_mesh = plsc.VectorSubcoreMesh(core_axis_name="c", subcore_axis_name="s")


CB1 = 5
NB1 = NCHW // CB1


def _k1_body(packed_hbm, deg_out, cnt_out,
             pv, rv, cv, vv, zv, deg_sh, cnt_sh, sem):
    cid = lax.axis_index("c")
    sid = lax.axis_index("s")
    wid = sid * NC + cid

    @pl.loop(0, ZR // L)
    def _zero(i):
        zv[pl.ds(i * L, L)] = jnp.zeros((L,), jnp.float32)

    pltpu.sync_copy(zv, deg_sh.at[pl.ds(sid * ZR, ZR)])
    pltpu.sync_copy(zv, cnt_sh.at[pl.ds(sid * ZR, ZR)])

    plsc.subcore_barrier()

    @pl.loop(0, NB1)
    def _blk(b):
        pltpu.sync_copy(packed_hbm.at[wid, b], pv)
        for q in range(CB1):
            for i in range(CH // L):
                sl = pl.ds(i * L, L)
                p = pv[q, sl]
                r = p >> SHIFT
                c = p & jnp.full((L,), MASK, jnp.int32)
                rv[q, sl] = r
                cv[q, sl] = c
                vv[q, sl] = jnp.where(
                    r != c, jnp.ones((L,), jnp.float32),
                    jnp.zeros((L,), jnp.float32))
        hs = []
        for q in range(CB1):
            hs.append(pltpu.async_copy(
                vv.at[q], deg_sh.at[rv.at[q]], sem, add=True))
            hs.append(pltpu.async_copy(
                vv.at[q], cnt_sh.at[cv.at[q]], sem, add=True))
        for h in hs:
            h.wait()

    plsc.subcore_barrier()

    pltpu.sync_copy(deg_sh.at[pl.ds(sid * ZR, ZR)], zv)
    pltpu.sync_copy(zv, deg_out.at[cid, 0, pl.ds(sid * ZR, ZR)])
    pltpu.sync_copy(cnt_sh.at[pl.ds(sid * ZR, ZR)], zv)
    pltpu.sync_copy(zv, cnt_out.at[cid, 0, pl.ds(sid * ZR, ZR)])


_k1 = pl.kernel(
    _k1_body,
    out_type=(
        jax.ShapeDtypeStruct((NC, 1, NPAD), jnp.float32),
        jax.ShapeDtypeStruct((NC, 1, NPAD), jnp.float32),
    ),
    mesh=_mesh,
    scratch_types=[
        pltpu.VMEM((CB1, CH), jnp.int32),
        pltpu.VMEM((CB1, CH), jnp.int32),
        pltpu.VMEM((CB1, CH), jnp.int32),
        pltpu.VMEM((CB1, CH), jnp.float32),
        pltpu.VMEM((ZR,), jnp.float32),
        pltpu.VMEM_SHARED((NPAD,), jnp.float32),
        pltpu.VMEM_SHARED((NPAD,), jnp.float32),
        pltpu.SemaphoreType.DMA,
    ],
)


NH = 5120
ACC_R = 6400
LDUMMY = NH
NCHW2 = E // NS // CH
CB2 = 25
NB2 = NCHW2 // CB2
WR2 = NH // NS


CH2 = 128
NBUF = 2
STEP = NBUF * CH2
CAP = 20224


def _k3_body(packed_hbm, xs_hbm, acc_out,
             pv, rc, rv, cv, tmp, acc_sh, gsems, ssems):
    cid = lax.axis_index("c")
    sid = lax.axis_index("s")

    @pl.loop(0, CAP // L)
    def _pf(i):
        rc[pl.ds(i * L, L)] = jnp.full((L,), LDUMMY, jnp.int32)

    @pl.loop(0, CH)
    def _zrow(r):
        for i in range(D // L):
            tmp[0, r, pl.ds(i * L, L)] = jnp.zeros((L,), jnp.float32)

    @pl.loop(0, WR2 // CH)
    def _zcp(k):
        pltpu.sync_copy(tmp.at[0, pl.ds(0, CH)],
                        acc_sh.at[pl.ds(sid * WR2 + k * CH, CH)])

    plsc.subcore_barrier()

    def _blk(b, n):
        pltpu.sync_copy(packed_hbm.at[sid, b], pv)
        for q in range(CB2):
            for i in range(CH // L):
                sl = pl.ds(i * L, L)
                p = pv[q, sl]
                r = p >> SHIFT
                c = p & jnp.full((L,), MASK, jnp.int32)
                lc = c - cid * NH
                ok = (r != c) & (lc >= 0) & (lc < NH)
                word = (r << SHIFT) | (lc & jnp.full((L,), MASK, jnp.int32))
                plsc.store_compressed(rc.at[pl.ds(n, L)], word, mask=ok)
                n = n + jnp.sum(jnp.where(ok, jnp.ones((L,), jnp.int32),
                                          jnp.zeros((L,), jnp.int32)))
        return n

    n = pl.loop(0, NB2, init_carry=jnp.int32(0))(_blk)
    nstep = (n + (STEP - 1)) // STEP

    @pl.loop(0, nstep)
    def _step(t):
        @pl.when(t > 0)
        def _w():
            for h in range(NBUF):
                pltpu.make_async_copy(
                    tmp.at[h], acc_sh.at[cv.at[h]], ssems.at[h]).wait()
        base = t * STEP
        for h in range(NBUF):
            for i in range(CH2 // L):
                w = rc[pl.ds(base + h * CH2 + i * L, L)]
                rv[h, pl.ds(i * L, L)] = w >> SHIFT
                cv[h, pl.ds(i * L, L)] = w & jnp.full((L,), MASK, jnp.int32)
        gh = [pltpu.async_copy(xs_hbm.at[rv.at[h]], tmp.at[h], gsems.at[h])
              for h in range(NBUF)]
        for h in range(NBUF):
            gh[h].wait()
            pltpu.async_copy(tmp.at[h], acc_sh.at[cv.at[h]], ssems.at[h],
                             add=True)

    @pl.when(nstep > 0)
    def _drain():
        for h in range(NBUF):
            pltpu.make_async_copy(
                tmp.at[h], acc_sh.at[cv.at[h]], ssems.at[h]).wait()

    plsc.subcore_barrier()

    @pl.loop(0, WR2 // CH)
    def _out(k):
        pltpu.sync_copy(acc_sh.at[pl.ds(sid * WR2 + k * CH, CH)],
                        tmp.at[0, pl.ds(0, CH)])
        pltpu.sync_copy(tmp.at[0, pl.ds(0, CH)],
                        acc_out.at[pl.ds(cid * NH + sid * WR2 + k * CH, CH)])


_k3 = pl.kernel(
    _k3_body,
    out_type=jax.ShapeDtypeStruct((NC * NH, D), jnp.float32),
    compiler_params=pltpu.CompilerParams(needs_layout_passes=False),
    mesh=_mesh,
    scratch_types=[
        pltpu.VMEM((CB2, CH), jnp.int32),
        pltpu.VMEM((CAP,), jnp.int32),
        pltpu.VMEM((NBUF, CH2), jnp.int32),
        pltpu.VMEM((NBUF, CH2), jnp.int32),
        pltpu.VMEM((NBUF, CH2, D), jnp.float32),
        pltpu.VMEM_SHARED((ACC_R, D), jnp.float32),
        pltpu.SemaphoreType.DMA((NBUF,)),
        pltpu.SemaphoreType.DMA((NBUF,)),
    ],
)


BLK = 1000


def _bcast(col):
    return lax.dot_general(col, jnp.ones((1, D), jnp.float32),
                           (((1,), (0,)), ((), ())),
                           preferred_element_type=jnp.float32)


def _k2_body(x_ref, deg_ref, cnt_ref, xs_ref, cinv_ref):
    dsum = deg_ref[0] + deg_ref[1] + 1.0
    csum = cnt_ref[0] + cnt_ref[1] + 1.0
    cinv_ref[...] = 1.0 / csum
    xs_ref[...] = x_ref[...] / _bcast(dsum)


def _k2(x, deg_col, cnt_col):
    return pl.pallas_call(
        _k2_body,
        grid=(N // BLK,),
        in_specs=[
            pl.BlockSpec((BLK, D), lambda i: (i, 0)),
            pl.BlockSpec((NC, BLK, 1), lambda i: (0, i, 0)),
            pl.BlockSpec((NC, BLK, 1), lambda i: (0, i, 0)),
        ],
        out_specs=[
            pl.BlockSpec((BLK, D), lambda i: (i, 0)),
            pl.BlockSpec((BLK, 1), lambda i: (i, 0)),
        ],
        out_shape=[
            jax.ShapeDtypeStruct((N, D), jnp.float32),
            jax.ShapeDtypeStruct((N, 1), jnp.float32),
        ],
    )(x, deg_col, cnt_col)


def _mm(a, b):
    return lax.dot_general(a, b, (((1,), (1,)), ((), ())),
                           preferred_element_type=jnp.float32)


def _k4_body(xs_ref, a_ref, cinv_ref,
             wc_ref, bc_ref, w1_ref, b1_ref, w2_ref, b2_ref, w3_ref, b3_ref,
             out_ref):
    agg = (xs_ref[...] + a_ref[...]) * _bcast(cinv_ref[...])
    h = jnp.maximum(_mm(agg, wc_ref[...]) + bc_ref[...], 0.0)
    h = jnp.maximum(_mm(h, w1_ref[...]) + b1_ref[...], 0.0)
    h = jnp.maximum(_mm(h, w2_ref[...]) + b2_ref[...], 0.0)
    out_ref[...] = jnp.sum(h * w3_ref[0], axis=1, keepdims=True) + b3_ref[0, 0]


def _k4(xs, acc_p, cinv, W_conv, b_conv, W1, b1, W2, b2, W3, b3):
    def full(s):
        return pl.BlockSpec(s, lambda i: (0,) * len(s))
    return pl.pallas_call(
        _k4_body,
        grid=(N // BLK,),
        in_specs=[
            pl.BlockSpec((BLK, D), lambda i: (i, 0)),
            pl.BlockSpec((BLK, D), lambda i: (i, 0)),
            pl.BlockSpec((BLK, 1), lambda i: (i, 0)),
            full((H1, D)), full((H1,)),
            full((H2, H1)), full((H2,)),
            full((H3, H2)), full((H3,)),
            full((1, H3)), pl.BlockSpec(memory_space=pltpu.SMEM),
        ],
        out_specs=pl.BlockSpec((BLK, 1), lambda i: (i, 0)),
        out_shape=jax.ShapeDtypeStruct((N, 1), jnp.float32),
    )(xs, acc_p, cinv, W_conv, b_conv, W1, b1, W2, b2, W3, b3.reshape(1, 1))


def kernel(x, edge_index, W_conv, b_conv, W1, b1, W2, b2, W3, b3):
    packed = (edge_index[0] << SHIFT) | edge_index[1]
    deg_p, cnt_p = _k1(packed.reshape(NW, NB1, CB1, CH))
    xs, cinv = _k2(x, deg_p.reshape(NC, NPAD, 1), cnt_p.reshape(NC, NPAD, 1))
    acc_p = _k3(packed.reshape(NS, NB2, CB2, CH), xs)
    return _k4(xs, acc_p, cinv, W_conv, b_conv, W1, b1, W2, b2, W3, b3)


# speedup vs baseline: 1.1609x; 1.1609x over previous
import jax
import jax.numpy as jnp
from jax import lax
from jax.experimental import pallas as pl
from jax.experimental.pallas import tpu as pltpu
from jax.experimental.pallas import tpu_sc as plsc

N = 10000
E = 320000
D = 128
H1, H2, H3 = 128, 64, 32

NC, NS, L = 2, 16, 16
NW = NC * NS
CH = 80
EW = E // NW
NCHW = EW // CH
NPAD = 10240
DUMMY = N
SHIFT = 14
MASK = (1 << SHIFT) - 1
ZR = NPAD // NS

_mesh = plsc.VectorSubcoreMesh(core_axis_name="c", subcore_axis_name="s")


CB1 = 5
NB1 = NCHW // CB1


def _k1_body(packed_hbm, deg_out, cnt_out,
             pv, rv, cv, vv, zv, deg_sh, cnt_sh, sem):
    cid = lax.axis_index("c")
    sid = lax.axis_index("s")
    wid = sid * NC + cid

    @pl.loop(0, ZR // L)
    def _zero(i):
        zv[pl.ds(i * L, L)] = jnp.zeros((L,), jnp.float32)

    pltpu.sync_copy(zv, deg_sh.at[pl.ds(sid * ZR, ZR)])
    pltpu.sync_copy(zv, cnt_sh.at[pl.ds(sid * ZR, ZR)])

    plsc.subcore_barrier()

    @pl.loop(0, NB1)
    def _blk(b):
        pltpu.sync_copy(packed_hbm.at[wid, b], pv)
        for q in range(CB1):
            for i in range(CH // L):
                sl = pl.ds(i * L, L)
                p = pv[q, sl]
                r = p >> SHIFT
                c = p & jnp.full((L,), MASK, jnp.int32)
                rv[q, sl] = r
                cv[q, sl] = c
                vv[q, sl] = jnp.where(
                    r != c, jnp.ones((L,), jnp.float32),
                    jnp.zeros((L,), jnp.float32))
        hs = []
        for q in range(CB1):
            hs.append(pltpu.async_copy(
                vv.at[q], deg_sh.at[rv.at[q]], sem, add=True))
            hs.append(pltpu.async_copy(
                vv.at[q], cnt_sh.at[cv.at[q]], sem, add=True))
        for h in hs:
            h.wait()

    plsc.subcore_barrier()

    pltpu.sync_copy(deg_sh.at[pl.ds(sid * ZR, ZR)], zv)
    pltpu.sync_copy(zv, deg_out.at[cid, 0, pl.ds(sid * ZR, ZR)])
    pltpu.sync_copy(cnt_sh.at[pl.ds(sid * ZR, ZR)], zv)
    pltpu.sync_copy(zv, cnt_out.at[cid, 0, pl.ds(sid * ZR, ZR)])


_k1 = pl.kernel(
    _k1_body,
    out_type=(
        jax.ShapeDtypeStruct((NC, 1, NPAD), jnp.float32),
        jax.ShapeDtypeStruct((NC, 1, NPAD), jnp.float32),
    ),
    mesh=_mesh,
    scratch_types=[
        pltpu.VMEM((CB1, CH), jnp.int32),
        pltpu.VMEM((CB1, CH), jnp.int32),
        pltpu.VMEM((CB1, CH), jnp.int32),
        pltpu.VMEM((CB1, CH), jnp.float32),
        pltpu.VMEM((ZR,), jnp.float32),
        pltpu.VMEM_SHARED((NPAD,), jnp.float32),
        pltpu.VMEM_SHARED((NPAD,), jnp.float32),
        pltpu.SemaphoreType.DMA,
    ],
)


NH = 5120
ACC_R = 6400
LDUMMY = NH
NCHW2 = E // NS // CH
CB2 = 25
NB2 = NCHW2 // CB2
WR2 = NH // NS


CH2 = 80
NBUF = 2
STEP = NBUF * CH2
CAP = 20160


def _k3_body(packed_hbm, xs_hbm, acc_out,
             pv, rc, rv, cv, tmp, acc_sh, gsems, ssems):
    cid = lax.axis_index("c")
    sid = lax.axis_index("s")

    @pl.loop(0, CAP // L)
    def _pf(i):
        rc[pl.ds(i * L, L)] = jnp.full((L,), LDUMMY, jnp.int32)

    @pl.loop(0, CH)
    def _zrow(r):
        for i in range(D // L):
            tmp[0, r, pl.ds(i * L, L)] = jnp.zeros((L,), jnp.float32)

    @pl.loop(0, WR2 // CH)
    def _zcp(k):
        pltpu.sync_copy(tmp.at[0, pl.ds(0, CH)],
                        acc_sh.at[pl.ds(sid * WR2 + k * CH, CH)])

    plsc.subcore_barrier()

    def _blk(b, n):
        pltpu.sync_copy(packed_hbm.at[sid, b], pv)
        for q in range(CB2):
            for i in range(CH // L):
                sl = pl.ds(i * L, L)
                p = pv[q, sl]
                r = p >> SHIFT
                c = p & jnp.full((L,), MASK, jnp.int32)
                lc = c - cid * NH
                ok = (r != c) & (lc >= 0) & (lc < NH)
                word = (r << SHIFT) | (lc & jnp.full((L,), MASK, jnp.int32))
                plsc.store_compressed(rc.at[pl.ds(n, L)], word, mask=ok)
                n = n + jnp.sum(jnp.where(ok, jnp.ones((L,), jnp.int32),
                                          jnp.zeros((L,), jnp.int32)))
        return n

    n = pl.loop(0, NB2, init_carry=jnp.int32(0))(_blk)
    nstep = (n + (STEP - 1)) // STEP

    @pl.loop(0, nstep)
    def _step(t):
        @pl.when(t > 0)
        def _w():
            for h in range(NBUF):
                pltpu.make_async_copy(
                    tmp.at[h], acc_sh.at[cv.at[h]], ssems.at[h]).wait()
        base = t * STEP
        for h in range(NBUF):
            for i in range(CH2 // L):
                w = rc[pl.ds(base + h * CH2 + i * L, L)]
                rv[h, pl.ds(i * L, L)] = w >> SHIFT
                cv[h, pl.ds(i * L, L)] = w & jnp.full((L,), MASK, jnp.int32)
        gh = [pltpu.async_copy(xs_hbm.at[rv.at[h]], tmp.at[h], gsems.at[h])
              for h in range(NBUF)]
        for h in range(NBUF):
            gh[h].wait()
            pltpu.async_copy(tmp.at[h], acc_sh.at[cv.at[h]], ssems.at[h],
                             add=True)

    @pl.when(nstep > 0)
    def _drain():
        for h in range(NBUF):
            pltpu.make_async_copy(
                tmp.at[h], acc_sh.at[cv.at[h]], ssems.at[h]).wait()

    plsc.subcore_barrier()

    @pl.loop(0, WR2 // CH)
    def _out(k):
        pltpu.sync_copy(acc_sh.at[pl.ds(sid * WR2 + k * CH, CH)],
                        tmp.at[0, pl.ds(0, CH)])
        pltpu.sync_copy(tmp.at[0, pl.ds(0, CH)],
                        acc_out.at[pl.ds(cid * NH + sid * WR2 + k * CH, CH)])


_k3 = pl.kernel(
    _k3_body,
    out_type=jax.ShapeDtypeStruct((NC * NH, D), jnp.float32),
    compiler_params=pltpu.CompilerParams(needs_layout_passes=False),
    mesh=_mesh,
    scratch_types=[
        pltpu.VMEM((CB2, CH), jnp.int32),
        pltpu.VMEM((CAP,), jnp.int32),
        pltpu.VMEM((NBUF, CH2), jnp.int32),
        pltpu.VMEM((NBUF, CH2), jnp.int32),
        pltpu.VMEM((NBUF, CH2, D), jnp.float32),
        pltpu.VMEM_SHARED((ACC_R, D), jnp.float32),
        pltpu.SemaphoreType.DMA((NBUF,)),
        pltpu.SemaphoreType.DMA((NBUF,)),
    ],
)


BLK = 1000


def _bcast(col):
    return lax.dot_general(col, jnp.ones((1, D), jnp.float32),
                           (((1,), (0,)), ((), ())),
                           preferred_element_type=jnp.float32)


def _k2_body(x_ref, deg_ref, cnt_ref, xs_ref, cinv_ref):
    dsum = deg_ref[0] + deg_ref[1] + 1.0
    csum = cnt_ref[0] + cnt_ref[1] + 1.0
    cinv_ref[...] = 1.0 / csum
    xs_ref[...] = x_ref[...] / _bcast(dsum)


def _k2(x, deg_col, cnt_col):
    return pl.pallas_call(
        _k2_body,
        grid=(N // BLK,),
        in_specs=[
            pl.BlockSpec((BLK, D), lambda i: (i, 0)),
            pl.BlockSpec((NC, BLK, 1), lambda i: (0, i, 0)),
            pl.BlockSpec((NC, BLK, 1), lambda i: (0, i, 0)),
        ],
        out_specs=[
            pl.BlockSpec((BLK, D), lambda i: (i, 0)),
            pl.BlockSpec((BLK, 1), lambda i: (i, 0)),
        ],
        out_shape=[
            jax.ShapeDtypeStruct((N, D), jnp.float32),
            jax.ShapeDtypeStruct((N, 1), jnp.float32),
        ],
    )(x, deg_col, cnt_col)


def _mm(a, b):
    return lax.dot_general(a, b, (((1,), (1,)), ((), ())),
                           preferred_element_type=jnp.float32)


def _k4_body(xs_ref, a_ref, cinv_ref,
             wc_ref, bc_ref, w1_ref, b1_ref, w2_ref, b2_ref, w3_ref, b3_ref,
             out_ref):
    agg = (xs_ref[...] + a_ref[...]) * _bcast(cinv_ref[...])
    h = jnp.maximum(_mm(agg, wc_ref[...]) + bc_ref[...], 0.0)
    h = jnp.maximum(_mm(h, w1_ref[...]) + b1_ref[...], 0.0)
    h = jnp.maximum(_mm(h, w2_ref[...]) + b2_ref[...], 0.0)
    out_ref[...] = jnp.sum(h * w3_ref[0], axis=1, keepdims=True) + b3_ref[0, 0]


def _k4(xs, acc_p, cinv, W_conv, b_conv, W1, b1, W2, b2, W3, b3):
    def full(s):
        return pl.BlockSpec(s, lambda i: (0,) * len(s))
    return pl.pallas_call(
        _k4_body,
        grid=(N // BLK,),
        in_specs=[
            pl.BlockSpec((BLK, D), lambda i: (i, 0)),
            pl.BlockSpec((BLK, D), lambda i: (i, 0)),
            pl.BlockSpec((BLK, 1), lambda i: (i, 0)),
            full((H1, D)), full((H1,)),
            full((H2, H1)), full((H2,)),
            full((H3, H2)), full((H3,)),
            full((1, H3)), pl.BlockSpec(memory_space=pltpu.SMEM),
        ],
        out_specs=pl.BlockSpec((BLK, 1), lambda i: (i, 0)),
        out_shape=jax.ShapeDtypeStruct((N, 1), jnp.float32),
    )(xs, acc_p, cinv, W_conv, b_conv, W1, b1, W2, b2, W3, b3.reshape(1, 1))


def kernel(x, edge_index, W_conv, b_conv, W1, b1, W2, b2, W3, b3):
    packed = (edge_index[0] << SHIFT) | edge_index[1]
    deg_p, cnt_p = _k1(packed.reshape(NW, NB1, CB1, CH))
    xs, cinv = _k2(x, deg_p.reshape(NC, NPAD, 1), cnt_p.reshape(NC, NPAD, 1))
    acc_p = _k3(packed.reshape(NS, NB2, CB2, CH), xs)
    return _k4(xs, acc_p, cinv, W_conv, b_conv, W1, b1, W2, b2, W3, b3)


# speedup vs baseline: 1.1635x; 1.0022x over previous
import jax
import jax.numpy as jnp
from jax import lax
from jax.experimental import pallas as pl
from jax.experimental.pallas import tpu as pltpu
from jax.experimental.pallas import tpu_sc as plsc

N = 10000
E = 320000
D = 128
H1, H2, H3 = 128, 64, 32

NC, NS, L = 2, 16, 16
NW = NC * NS
CH = 80
EW = E // NW
NCHW = EW // CH
NPAD = 10240
DUMMY = N
SHIFT = 14
MASK = (1 << SHIFT) - 1
ZR = NPAD // NS

_mesh = plsc.VectorSubcoreMesh(core_axis_name="c", subcore_axis_name="s")


CB1 = 5
NB1 = NCHW // CB1


def _k1_body(packed_hbm, deg_out, cnt_out,
             pv, rv, cv, vv, zv, deg_sh, cnt_sh, sem):
    cid = lax.axis_index("c")
    sid = lax.axis_index("s")
    wid = sid * NC + cid

    @pl.loop(0, ZR // L)
    def _zero(i):
        zv[pl.ds(i * L, L)] = jnp.zeros((L,), jnp.float32)

    pltpu.sync_copy(zv, deg_sh.at[pl.ds(sid * ZR, ZR)])
    pltpu.sync_copy(zv, cnt_sh.at[pl.ds(sid * ZR, ZR)])

    plsc.subcore_barrier()

    @pl.loop(0, NB1)
    def _blk(b):
        pltpu.sync_copy(packed_hbm.at[wid, b], pv)
        for q in range(CB1):
            for i in range(CH // L):
                sl = pl.ds(i * L, L)
                p = pv[q, sl]
                r = p >> SHIFT
                c = p & jnp.full((L,), MASK, jnp.int32)
                rv[q, sl] = r
                cv[q, sl] = c
                vv[q, sl] = jnp.where(
                    r != c, jnp.ones((L,), jnp.float32),
                    jnp.zeros((L,), jnp.float32))
        hs = []
        for q in range(CB1):
            hs.append(pltpu.async_copy(
                vv.at[q], deg_sh.at[rv.at[q]], sem, add=True))
            hs.append(pltpu.async_copy(
                vv.at[q], cnt_sh.at[cv.at[q]], sem, add=True))
        for h in hs:
            h.wait()

    plsc.subcore_barrier()

    pltpu.sync_copy(deg_sh.at[pl.ds(sid * ZR, ZR)], zv)
    pltpu.sync_copy(zv, deg_out.at[cid, 0, pl.ds(sid * ZR, ZR)])
    pltpu.sync_copy(cnt_sh.at[pl.ds(sid * ZR, ZR)], zv)
    pltpu.sync_copy(zv, cnt_out.at[cid, 0, pl.ds(sid * ZR, ZR)])


_k1 = pl.kernel(
    _k1_body,
    out_type=(
        jax.ShapeDtypeStruct((NC, 1, NPAD), jnp.float32),
        jax.ShapeDtypeStruct((NC, 1, NPAD), jnp.float32),
    ),
    mesh=_mesh,
    scratch_types=[
        pltpu.VMEM((CB1, CH), jnp.int32),
        pltpu.VMEM((CB1, CH), jnp.int32),
        pltpu.VMEM((CB1, CH), jnp.int32),
        pltpu.VMEM((CB1, CH), jnp.float32),
        pltpu.VMEM((ZR,), jnp.float32),
        pltpu.VMEM_SHARED((NPAD,), jnp.float32),
        pltpu.VMEM_SHARED((NPAD,), jnp.float32),
        pltpu.SemaphoreType.DMA,
    ],
)


NH = 5120
ACC_R = 6400
LDUMMY = NH
NCHW2 = E // NS // CH
CB2 = 25
NB2 = NCHW2 // CB2
WR2 = NH // NS


CH2 = 64
NBUF = 2
STEP = NBUF * CH2
CAP = 20096
WCH = CH2 if WR2 % CH2 == 0 else 80


def _k3_body(packed_hbm, xs_hbm, acc_out,
             pv, rc, rv, cv, tmp, acc_sh, gsems, ssems):
    cid = lax.axis_index("c")
    sid = lax.axis_index("s")

    @pl.loop(0, CAP // L)
    def _pf(i):
        rc[pl.ds(i * L, L)] = jnp.full((L,), LDUMMY, jnp.int32)

    @pl.loop(0, WCH)
    def _zrow(r):
        for i in range(D // L):
            tmp[0, r, pl.ds(i * L, L)] = jnp.zeros((L,), jnp.float32)

    @pl.loop(0, WR2 // WCH)
    def _zcp(k):
        pltpu.sync_copy(tmp.at[0, pl.ds(0, WCH)],
                        acc_sh.at[pl.ds(sid * WR2 + k * WCH, WCH)])

    plsc.subcore_barrier()

    def _blk(b, n):
        pltpu.sync_copy(packed_hbm.at[sid, b], pv)
        for q in range(CB2):
            for i in range(CH // L):
                sl = pl.ds(i * L, L)
                p = pv[q, sl]
                r = p >> SHIFT
                c = p & jnp.full((L,), MASK, jnp.int32)
                lc = c - cid * NH
                ok = (r != c) & (lc >= 0) & (lc < NH)
                word = (r << SHIFT) | (lc & jnp.full((L,), MASK, jnp.int32))
                plsc.store_compressed(rc.at[pl.ds(n, L)], word, mask=ok)
                n = n + jnp.sum(jnp.where(ok, jnp.ones((L,), jnp.int32),
                                          jnp.zeros((L,), jnp.int32)))
        return n

    n = pl.loop(0, NB2, init_carry=jnp.int32(0))(_blk)
    nstep = (n + (STEP - 1)) // STEP

    @pl.loop(0, nstep)
    def _step(t):
        @pl.when(t > 0)
        def _w():
            for h in range(NBUF):
                pltpu.make_async_copy(
                    tmp.at[h], acc_sh.at[cv.at[h]], ssems.at[h]).wait()
        base = t * STEP
        for h in range(NBUF):
            for i in range(CH2 // L):
                w = rc[pl.ds(base + h * CH2 + i * L, L)]
                rv[h, pl.ds(i * L, L)] = w >> SHIFT
                cv[h, pl.ds(i * L, L)] = w & jnp.full((L,), MASK, jnp.int32)
        gh = [pltpu.async_copy(xs_hbm.at[rv.at[h]], tmp.at[h], gsems.at[h])
              for h in range(NBUF)]
        for h in range(NBUF):
            gh[h].wait()
            pltpu.async_copy(tmp.at[h], acc_sh.at[cv.at[h]], ssems.at[h],
                             add=True)

    @pl.when(nstep > 0)
    def _drain():
        for h in range(NBUF):
            pltpu.make_async_copy(
                tmp.at[h], acc_sh.at[cv.at[h]], ssems.at[h]).wait()

    plsc.subcore_barrier()

    @pl.loop(0, WR2 // WCH)
    def _out(k):
        pltpu.sync_copy(acc_sh.at[pl.ds(sid * WR2 + k * WCH, WCH)],
                        tmp.at[0, pl.ds(0, WCH)])
        pltpu.sync_copy(tmp.at[0, pl.ds(0, WCH)],
                        acc_out.at[pl.ds(cid * NH + sid * WR2 + k * WCH, WCH)])


_k3 = pl.kernel(
    _k3_body,
    out_type=jax.ShapeDtypeStruct((NC * NH, D), jnp.float32),
    compiler_params=pltpu.CompilerParams(needs_layout_passes=False),
    mesh=_mesh,
    scratch_types=[
        pltpu.VMEM((CB2, CH), jnp.int32),
        pltpu.VMEM((CAP,), jnp.int32),
        pltpu.VMEM((NBUF, CH2), jnp.int32),
        pltpu.VMEM((NBUF, CH2), jnp.int32),
        pltpu.VMEM((NBUF, CH2, D), jnp.float32),
        pltpu.VMEM_SHARED((ACC_R, D), jnp.float32),
        pltpu.SemaphoreType.DMA((NBUF,)),
        pltpu.SemaphoreType.DMA((NBUF,)),
    ],
)


BLK = 1000


def _bcast(col):
    return lax.dot_general(col, jnp.ones((1, D), jnp.float32),
                           (((1,), (0,)), ((), ())),
                           preferred_element_type=jnp.float32)


def _k2_body(x_ref, deg_ref, cnt_ref, xs_ref, cinv_ref):
    dsum = deg_ref[0] + deg_ref[1] + 1.0
    csum = cnt_ref[0] + cnt_ref[1] + 1.0
    cinv_ref[...] = 1.0 / csum
    xs_ref[...] = x_ref[...] / _bcast(dsum)


def _k2(x, deg_col, cnt_col):
    return pl.pallas_call(
        _k2_body,
        grid=(N // BLK,),
        in_specs=[
            pl.BlockSpec((BLK, D), lambda i: (i, 0)),
            pl.BlockSpec((NC, BLK, 1), lambda i: (0, i, 0)),
            pl.BlockSpec((NC, BLK, 1), lambda i: (0, i, 0)),
        ],
        out_specs=[
            pl.BlockSpec((BLK, D), lambda i: (i, 0)),
            pl.BlockSpec((BLK, 1), lambda i: (i, 0)),
        ],
        out_shape=[
            jax.ShapeDtypeStruct((N, D), jnp.float32),
            jax.ShapeDtypeStruct((N, 1), jnp.float32),
        ],
    )(x, deg_col, cnt_col)


def _mm(a, b):
    return lax.dot_general(a, b, (((1,), (1,)), ((), ())),
                           preferred_element_type=jnp.float32)


def _k4_body(xs_ref, a_ref, cinv_ref,
             wc_ref, bc_ref, w1_ref, b1_ref, w2_ref, b2_ref, w3_ref, b3_ref,
             out_ref):
    agg = (xs_ref[...] + a_ref[...]) * _bcast(cinv_ref[...])
    h = jnp.maximum(_mm(agg, wc_ref[...]) + bc_ref[...], 0.0)
    h = jnp.maximum(_mm(h, w1_ref[...]) + b1_ref[...], 0.0)
    h = jnp.maximum(_mm(h, w2_ref[...]) + b2_ref[...], 0.0)
    out_ref[...] = jnp.sum(h * w3_ref[0], axis=1, keepdims=True) + b3_ref[0, 0]


def _k4(xs, acc_p, cinv, W_conv, b_conv, W1, b1, W2, b2, W3, b3):
    def full(s):
        return pl.BlockSpec(s, lambda i: (0,) * len(s))
    return pl.pallas_call(
        _k4_body,
        grid=(N // BLK,),
        in_specs=[
            pl.BlockSpec((BLK, D), lambda i: (i, 0)),
            pl.BlockSpec((BLK, D), lambda i: (i, 0)),
            pl.BlockSpec((BLK, 1), lambda i: (i, 0)),
            full((H1, D)), full((H1,)),
            full((H2, H1)), full((H2,)),
            full((H3, H2)), full((H3,)),
            full((1, H3)), pl.BlockSpec(memory_space=pltpu.SMEM),
        ],
        out_specs=pl.BlockSpec((BLK, 1), lambda i: (i, 0)),
        out_shape=jax.ShapeDtypeStruct((N, 1), jnp.float32),
    )(xs, acc_p, cinv, W_conv, b_conv, W1, b1, W2, b2, W3, b3.reshape(1, 1))


def kernel(x, edge_index, W_conv, b_conv, W1, b1, W2, b2, W3, b3):
    packed = (edge_index[0] << SHIFT) | edge_index[1]
    deg_p, cnt_p = _k1(packed.reshape(NW, NB1, CB1, CH))
    xs, cinv = _k2(x, deg_p.reshape(NC, NPAD, 1), cnt_p.reshape(NC, NPAD, 1))
    acc_p = _k3(packed.reshape(NS, NB2, CB2, CH), xs)
    return _k4(xs, acc_p, cinv, W_conv, b_conv, W1, b1, W2, b2, W3, b3)


# speedup vs baseline: 1.1637x; 1.0002x over previous
import jax
import jax.numpy as jnp
from jax import lax
from jax.experimental import pallas as pl
from jax.experimental.pallas import tpu as pltpu
from jax.experimental.pallas import tpu_sc as plsc

N = 10000
E = 320000
D = 128
H1, H2, H3 = 128, 64, 32

NC, NS, L = 2, 16, 16
NW = NC * NS
CH = 80
EW = E // NW
NCHW = EW // CH
NPAD = 10240
DUMMY = N
SHIFT = 14
MASK = (1 << SHIFT) - 1
ZR = NPAD // NS

_mesh = plsc.VectorSubcoreMesh(core_axis_name="c", subcore_axis_name="s")


CB1 = 5
NB1 = NCHW // CB1


def _k1_body(packed_hbm, deg_out, cnt_out,
             pv, rv, cv, vv, zv, deg_sh, cnt_sh, sem):
    cid = lax.axis_index("c")
    sid = lax.axis_index("s")
    wid = sid * NC + cid

    @pl.loop(0, ZR // L)
    def _zero(i):
        zv[pl.ds(i * L, L)] = jnp.zeros((L,), jnp.float32)

    pltpu.sync_copy(zv, deg_sh.at[pl.ds(sid * ZR, ZR)])
    pltpu.sync_copy(zv, cnt_sh.at[pl.ds(sid * ZR, ZR)])

    plsc.subcore_barrier()

    @pl.loop(0, NB1)
    def _blk(b):
        pltpu.sync_copy(packed_hbm.at[wid, b], pv)
        for q in range(CB1):
            for i in range(CH // L):
                sl = pl.ds(i * L, L)
                p = pv[q, sl]
                r = p >> SHIFT
                c = p & jnp.full((L,), MASK, jnp.int32)
                rv[q, sl] = r
                cv[q, sl] = c
                vv[q, sl] = jnp.where(
                    r != c, jnp.ones((L,), jnp.float32),
                    jnp.zeros((L,), jnp.float32))
        hs = []
        for q in range(CB1):
            hs.append(pltpu.async_copy(
                vv.at[q], deg_sh.at[rv.at[q]], sem, add=True))
            hs.append(pltpu.async_copy(
                vv.at[q], cnt_sh.at[cv.at[q]], sem, add=True))
        for h in hs:
            h.wait()

    plsc.subcore_barrier()

    pltpu.sync_copy(deg_sh.at[pl.ds(sid * ZR, ZR)], zv)
    pltpu.sync_copy(zv, deg_out.at[cid, 0, pl.ds(sid * ZR, ZR)])
    pltpu.sync_copy(cnt_sh.at[pl.ds(sid * ZR, ZR)], zv)
    pltpu.sync_copy(zv, cnt_out.at[cid, 0, pl.ds(sid * ZR, ZR)])


_k1 = pl.kernel(
    _k1_body,
    out_type=(
        jax.ShapeDtypeStruct((NC, 1, NPAD), jnp.float32),
        jax.ShapeDtypeStruct((NC, 1, NPAD), jnp.float32),
    ),
    compiler_params=pltpu.CompilerParams(skip_device_barrier=True),
    mesh=_mesh,
    scratch_types=[
        pltpu.VMEM((CB1, CH), jnp.int32),
        pltpu.VMEM((CB1, CH), jnp.int32),
        pltpu.VMEM((CB1, CH), jnp.int32),
        pltpu.VMEM((CB1, CH), jnp.float32),
        pltpu.VMEM((ZR,), jnp.float32),
        pltpu.VMEM_SHARED((NPAD,), jnp.float32),
        pltpu.VMEM_SHARED((NPAD,), jnp.float32),
        pltpu.SemaphoreType.DMA,
    ],
)


NH = 5120
ACC_R = 6400
LDUMMY = NH
NCHW2 = E // NS // CH
CB2 = 25
NB2 = NCHW2 // CB2
WR2 = NH // NS


CH2 = 64
NBUF = 2
STEP = NBUF * CH2
CAP = 20096
WCH = CH2 if WR2 % CH2 == 0 else 80


def _k3_body(packed_hbm, xs_hbm, acc_out,
             pv, rc, rv, cv, tmp, acc_sh, gsems, ssems):
    cid = lax.axis_index("c")
    sid = lax.axis_index("s")

    @pl.loop(0, CAP // L)
    def _pf(i):
        rc[pl.ds(i * L, L)] = jnp.full((L,), LDUMMY, jnp.int32)

    @pl.loop(0, WCH)
    def _zrow(r):
        for i in range(D // L):
            tmp[0, r, pl.ds(i * L, L)] = jnp.zeros((L,), jnp.float32)

    @pl.loop(0, WR2 // WCH)
    def _zcp(k):
        pltpu.sync_copy(tmp.at[0, pl.ds(0, WCH)],
                        acc_sh.at[pl.ds(sid * WR2 + k * WCH, WCH)])

    plsc.subcore_barrier()

    def _blk(b, n):
        pltpu.sync_copy(packed_hbm.at[sid, b], pv)
        for q in range(CB2):
            for i in range(CH // L):
                sl = pl.ds(i * L, L)
                p = pv[q, sl]
                r = p >> SHIFT
                c = p & jnp.full((L,), MASK, jnp.int32)
                lc = c - cid * NH
                ok = (r != c) & (lc >= 0) & (lc < NH)
                word = (r << SHIFT) | (lc & jnp.full((L,), MASK, jnp.int32))
                plsc.store_compressed(rc.at[pl.ds(n, L)], word, mask=ok)
                n = n + jnp.sum(jnp.where(ok, jnp.ones((L,), jnp.int32),
                                          jnp.zeros((L,), jnp.int32)))
        return n

    n = pl.loop(0, NB2, init_carry=jnp.int32(0))(_blk)
    nstep = (n + (STEP - 1)) // STEP

    @pl.loop(0, nstep)
    def _step(t):
        @pl.when(t > 0)
        def _w():
            for h in range(NBUF):
                pltpu.make_async_copy(
                    tmp.at[h], acc_sh.at[cv.at[h]], ssems.at[h]).wait()
        base = t * STEP
        for h in range(NBUF):
            for i in range(CH2 // L):
                w = rc[pl.ds(base + h * CH2 + i * L, L)]
                rv[h, pl.ds(i * L, L)] = w >> SHIFT
                cv[h, pl.ds(i * L, L)] = w & jnp.full((L,), MASK, jnp.int32)
        gh = [pltpu.async_copy(xs_hbm.at[rv.at[h]], tmp.at[h], gsems.at[h])
              for h in range(NBUF)]
        for h in range(NBUF):
            gh[h].wait()
            pltpu.async_copy(tmp.at[h], acc_sh.at[cv.at[h]], ssems.at[h],
                             add=True)

    @pl.when(nstep > 0)
    def _drain():
        for h in range(NBUF):
            pltpu.make_async_copy(
                tmp.at[h], acc_sh.at[cv.at[h]], ssems.at[h]).wait()

    plsc.subcore_barrier()

    @pl.loop(0, WR2 // WCH)
    def _out(k):
        pltpu.sync_copy(acc_sh.at[pl.ds(sid * WR2 + k * WCH, WCH)],
                        tmp.at[0, pl.ds(0, WCH)])
        pltpu.sync_copy(tmp.at[0, pl.ds(0, WCH)],
                        acc_out.at[pl.ds(cid * NH + sid * WR2 + k * WCH, WCH)])


_k3 = pl.kernel(
    _k3_body,
    out_type=jax.ShapeDtypeStruct((NC * NH, D), jnp.float32),
    compiler_params=pltpu.CompilerParams(needs_layout_passes=False,
                                         skip_device_barrier=True),
    mesh=_mesh,
    scratch_types=[
        pltpu.VMEM((CB2, CH), jnp.int32),
        pltpu.VMEM((CAP,), jnp.int32),
        pltpu.VMEM((NBUF, CH2), jnp.int32),
        pltpu.VMEM((NBUF, CH2), jnp.int32),
        pltpu.VMEM((NBUF, CH2, D), jnp.float32),
        pltpu.VMEM_SHARED((ACC_R, D), jnp.float32),
        pltpu.SemaphoreType.DMA((NBUF,)),
        pltpu.SemaphoreType.DMA((NBUF,)),
    ],
)


BLK = 1000


def _bcast(col):
    return lax.dot_general(col, jnp.ones((1, D), jnp.float32),
                           (((1,), (0,)), ((), ())),
                           preferred_element_type=jnp.float32)


def _k2_body(x_ref, deg_ref, cnt_ref, xs_ref, cinv_ref):
    dsum = deg_ref[0] + deg_ref[1] + 1.0
    csum = cnt_ref[0] + cnt_ref[1] + 1.0
    cinv_ref[...] = 1.0 / csum
    xs_ref[...] = x_ref[...] / _bcast(dsum)


def _k2(x, deg_col, cnt_col):
    return pl.pallas_call(
        _k2_body,
        grid=(N // BLK,),
        in_specs=[
            pl.BlockSpec((BLK, D), lambda i: (i, 0)),
            pl.BlockSpec((NC, BLK, 1), lambda i: (0, i, 0)),
            pl.BlockSpec((NC, BLK, 1), lambda i: (0, i, 0)),
        ],
        out_specs=[
            pl.BlockSpec((BLK, D), lambda i: (i, 0)),
            pl.BlockSpec((BLK, 1), lambda i: (i, 0)),
        ],
        out_shape=[
            jax.ShapeDtypeStruct((N, D), jnp.float32),
            jax.ShapeDtypeStruct((N, 1), jnp.float32),
        ],
    )(x, deg_col, cnt_col)


def _mm(a, b):
    return lax.dot_general(a, b, (((1,), (1,)), ((), ())),
                           preferred_element_type=jnp.float32)


def _k4_body(xs_ref, a_ref, cinv_ref,
             wc_ref, bc_ref, w1_ref, b1_ref, w2_ref, b2_ref, w3_ref, b3_ref,
             out_ref):
    agg = (xs_ref[...] + a_ref[...]) * _bcast(cinv_ref[...])
    h = jnp.maximum(_mm(agg, wc_ref[...]) + bc_ref[...], 0.0)
    h = jnp.maximum(_mm(h, w1_ref[...]) + b1_ref[...], 0.0)
    h = jnp.maximum(_mm(h, w2_ref[...]) + b2_ref[...], 0.0)
    out_ref[...] = jnp.sum(h * w3_ref[0], axis=1, keepdims=True) + b3_ref[0, 0]


def _k4(xs, acc_p, cinv, W_conv, b_conv, W1, b1, W2, b2, W3, b3):
    def full(s):
        return pl.BlockSpec(s, lambda i: (0,) * len(s))
    return pl.pallas_call(
        _k4_body,
        grid=(N // BLK,),
        in_specs=[
            pl.BlockSpec((BLK, D), lambda i: (i, 0)),
            pl.BlockSpec((BLK, D), lambda i: (i, 0)),
            pl.BlockSpec((BLK, 1), lambda i: (i, 0)),
            full((H1, D)), full((H1,)),
            full((H2, H1)), full((H2,)),
            full((H3, H2)), full((H3,)),
            full((1, H3)), pl.BlockSpec(memory_space=pltpu.SMEM),
        ],
        out_specs=pl.BlockSpec((BLK, 1), lambda i: (i, 0)),
        out_shape=jax.ShapeDtypeStruct((N, 1), jnp.float32),
    )(xs, acc_p, cinv, W_conv, b_conv, W1, b1, W2, b2, W3, b3.reshape(1, 1))


def kernel(x, edge_index, W_conv, b_conv, W1, b1, W2, b2, W3, b3):
    packed = (edge_index[0] << SHIFT) | edge_index[1]
    deg_p, cnt_p = _k1(packed.reshape(NW, NB1, CB1, CH))
    xs, cinv = _k2(x, deg_p.reshape(NC, NPAD, 1), cnt_p.reshape(NC, NPAD, 1))
    acc_p = _k3(packed.reshape(NS, NB2, CB2, CH), xs)
    return _k4(xs, acc_p, cinv, W_conv, b_conv, W1, b1, W2, b2, W3, b3)


# speedup vs baseline: 1.1638x; 1.0000x over previous
import jax
import jax.numpy as jnp
from jax import lax
from jax.experimental import pallas as pl
from jax.experimental.pallas import tpu as pltpu
from jax.experimental.pallas import tpu_sc as plsc

N = 10000
E = 320000
D = 128
H1, H2, H3 = 128, 64, 32

NC, NS, L = 2, 16, 16
NW = NC * NS
CH = 80
EW = E // NW
NCHW = EW // CH
NPAD = 10240
SHIFT = 14
MASK = (1 << SHIFT) - 1
ZR = NPAD // NS

_mesh = plsc.VectorSubcoreMesh(core_axis_name="c", subcore_axis_name="s")


CB1 = 5
NB1 = NCHW // CB1


def _k1_body(packed_hbm, deg_out, cnt_out,
             pv, rv, cv, vv, zv, deg_sh, cnt_sh, sem):
    cid = lax.axis_index("c")
    sid = lax.axis_index("s")
    wid = sid * NC + cid

    @pl.loop(0, ZR // L)
    def _zero(i):
        zv[pl.ds(i * L, L)] = jnp.zeros((L,), jnp.float32)

    pltpu.sync_copy(zv, deg_sh.at[pl.ds(sid * ZR, ZR)])
    pltpu.sync_copy(zv, cnt_sh.at[pl.ds(sid * ZR, ZR)])

    plsc.subcore_barrier()

    @pl.loop(0, NB1)
    def _blk(b):
        pltpu.sync_copy(packed_hbm.at[wid, b], pv)
        for q in range(CB1):
            for i in range(CH // L):
                sl = pl.ds(i * L, L)
                p = pv[q, sl]
                r = p >> SHIFT
                c = p & jnp.full((L,), MASK, jnp.int32)
                rv[q, sl] = r
                cv[q, sl] = c
                vv[q, sl] = jnp.where(
                    r != c, jnp.ones((L,), jnp.float32),
                    jnp.zeros((L,), jnp.float32))
        hs = []
        for q in range(CB1):
            hs.append(pltpu.async_copy(
                vv.at[q], deg_sh.at[rv.at[q]], sem, add=True))
            hs.append(pltpu.async_copy(
                vv.at[q], cnt_sh.at[cv.at[q]], sem, add=True))
        for h in hs:
            h.wait()

    plsc.subcore_barrier()

    pltpu.sync_copy(deg_sh.at[pl.ds(sid * ZR, ZR)], zv)
    pltpu.sync_copy(zv, deg_out.at[cid, 0, pl.ds(sid * ZR, ZR)])
    pltpu.sync_copy(cnt_sh.at[pl.ds(sid * ZR, ZR)], zv)
    pltpu.sync_copy(zv, cnt_out.at[cid, 0, pl.ds(sid * ZR, ZR)])


_k1 = pl.kernel(
    _k1_body,
    out_type=(
        jax.ShapeDtypeStruct((NC, 1, NPAD), jnp.float32),
        jax.ShapeDtypeStruct((NC, 1, NPAD), jnp.float32),
    ),
    mesh=_mesh,
    scratch_types=[
        pltpu.VMEM((CB1, CH), jnp.int32),
        pltpu.VMEM((CB1, CH), jnp.int32),
        pltpu.VMEM((CB1, CH), jnp.int32),
        pltpu.VMEM((CB1, CH), jnp.float32),
        pltpu.VMEM((ZR,), jnp.float32),
        pltpu.VMEM_SHARED((NPAD,), jnp.float32),
        pltpu.VMEM_SHARED((NPAD,), jnp.float32),
        pltpu.SemaphoreType.DMA,
    ],
)


NH = 5120
ACC_R = 6400
LDUMMY = NH
NCHW2 = E // NS // CH
CB2 = 25
NB2 = NCHW2 // CB2
WR2 = NH // NS


CH2 = 64
NBUF = 2
STEP = NBUF * CH2
CAP = 20096
WCH = CH2 if WR2 % CH2 == 0 else 80


def _k3_body(packed_hbm, xs_hbm, acc_out,
             pv, rc, rv, cv, tmp, acc_sh, gsems, ssems):
    cid = lax.axis_index("c")
    sid = lax.axis_index("s")

    @pl.loop(0, CAP // L)
    def _pf(i):
        rc[pl.ds(i * L, L)] = jnp.full((L,), LDUMMY, jnp.int32)

    @pl.loop(0, WCH)
    def _zrow(r):
        for i in range(D // L):
            tmp[0, r, pl.ds(i * L, L)] = jnp.zeros((L,), jnp.float32)

    @pl.loop(0, WR2 // WCH)
    def _zcp(k):
        pltpu.sync_copy(tmp.at[0, pl.ds(0, WCH)],
                        acc_sh.at[pl.ds(sid * WR2 + k * WCH, WCH)])

    plsc.subcore_barrier()

    def _blk(b, n):
        pltpu.sync_copy(packed_hbm.at[sid, b], pv)
        for q in range(CB2):
            for i in range(CH // L):
                sl = pl.ds(i * L, L)
                p = pv[q, sl]
                r = p >> SHIFT
                c = p & jnp.full((L,), MASK, jnp.int32)
                lc = c - cid * NH
                ok = (r != c) & (lc >= 0) & (lc < NH)
                word = (r << SHIFT) | (lc & jnp.full((L,), MASK, jnp.int32))
                plsc.store_compressed(rc.at[pl.ds(n, L)], word, mask=ok)
                n = n + jnp.sum(jnp.where(ok, jnp.ones((L,), jnp.int32),
                                          jnp.zeros((L,), jnp.int32)))
        return n

    n = pl.loop(0, NB2, init_carry=jnp.int32(0))(_blk)
    nstep = (n + (STEP - 1)) // STEP

    @pl.loop(0, nstep)
    def _step(t):
        @pl.when(t > 0)
        def _w():
            for h in range(NBUF):
                pltpu.make_async_copy(
                    tmp.at[h], acc_sh.at[cv.at[h]], ssems.at[h]).wait()
        base = t * STEP
        for h in range(NBUF):
            for i in range(CH2 // L):
                w = rc[pl.ds(base + h * CH2 + i * L, L)]
                rv[h, pl.ds(i * L, L)] = w >> SHIFT
                cv[h, pl.ds(i * L, L)] = w & jnp.full((L,), MASK, jnp.int32)
        gh = [pltpu.async_copy(xs_hbm.at[rv.at[h]], tmp.at[h], gsems.at[h])
              for h in range(NBUF)]
        for h in range(NBUF):
            gh[h].wait()
            pltpu.async_copy(tmp.at[h], acc_sh.at[cv.at[h]], ssems.at[h],
                             add=True)

    @pl.when(nstep > 0)
    def _drain():
        for h in range(NBUF):
            pltpu.make_async_copy(
                tmp.at[h], acc_sh.at[cv.at[h]], ssems.at[h]).wait()

    plsc.subcore_barrier()

    @pl.loop(0, WR2 // WCH)
    def _out(k):
        pltpu.sync_copy(acc_sh.at[pl.ds(sid * WR2 + k * WCH, WCH)],
                        tmp.at[0, pl.ds(0, WCH)])
        pltpu.sync_copy(tmp.at[0, pl.ds(0, WCH)],
                        acc_out.at[pl.ds(cid * NH + sid * WR2 + k * WCH, WCH)])


_k3 = pl.kernel(
    _k3_body,
    out_type=jax.ShapeDtypeStruct((NC * NH, D), jnp.float32),
    compiler_params=pltpu.CompilerParams(needs_layout_passes=False),
    mesh=_mesh,
    scratch_types=[
        pltpu.VMEM((CB2, CH), jnp.int32),
        pltpu.VMEM((CAP,), jnp.int32),
        pltpu.VMEM((NBUF, CH2), jnp.int32),
        pltpu.VMEM((NBUF, CH2), jnp.int32),
        pltpu.VMEM((NBUF, CH2, D), jnp.float32),
        pltpu.VMEM_SHARED((ACC_R, D), jnp.float32),
        pltpu.SemaphoreType.DMA((NBUF,)),
        pltpu.SemaphoreType.DMA((NBUF,)),
    ],
)


BLK = 1000


def _bcast(col):
    return lax.dot_general(col, jnp.ones((1, D), jnp.float32),
                           (((1,), (0,)), ((), ())),
                           preferred_element_type=jnp.float32)


def _k2_body(x_ref, deg_ref, cnt_ref, xs_ref, cinv_ref):
    dsum = deg_ref[0] + deg_ref[1] + 1.0
    csum = cnt_ref[0] + cnt_ref[1] + 1.0
    cinv_ref[...] = 1.0 / csum
    xs_ref[...] = x_ref[...] / _bcast(dsum)


def _k2(x, deg_col, cnt_col):
    return pl.pallas_call(
        _k2_body,
        grid=(N // BLK,),
        in_specs=[
            pl.BlockSpec((BLK, D), lambda i: (i, 0)),
            pl.BlockSpec((NC, BLK, 1), lambda i: (0, i, 0)),
            pl.BlockSpec((NC, BLK, 1), lambda i: (0, i, 0)),
        ],
        out_specs=[
            pl.BlockSpec((BLK, D), lambda i: (i, 0)),
            pl.BlockSpec((BLK, 1), lambda i: (i, 0)),
        ],
        out_shape=[
            jax.ShapeDtypeStruct((N, D), jnp.float32),
            jax.ShapeDtypeStruct((N, 1), jnp.float32),
        ],
    )(x, deg_col, cnt_col)


def _mm(a, b):
    return lax.dot_general(a, b, (((1,), (1,)), ((), ())),
                           preferred_element_type=jnp.float32)


def _k4_body(xs_ref, a_ref, cinv_ref,
             wc_ref, bc_ref, w1_ref, b1_ref, w2_ref, b2_ref, w3_ref, b3_ref,
             out_ref):
    agg = (xs_ref[...] + a_ref[...]) * _bcast(cinv_ref[...])
    h = jnp.maximum(_mm(agg, wc_ref[...]) + bc_ref[...], 0.0)
    h = jnp.maximum(_mm(h, w1_ref[...]) + b1_ref[...], 0.0)
    h = jnp.maximum(_mm(h, w2_ref[...]) + b2_ref[...], 0.0)
    out_ref[...] = jnp.sum(h * w3_ref[0], axis=1, keepdims=True) + b3_ref[0, 0]


def _k4(xs, acc_p, cinv, W_conv, b_conv, W1, b1, W2, b2, W3, b3):
    def full(s):
        return pl.BlockSpec(s, lambda i: (0,) * len(s))
    return pl.pallas_call(
        _k4_body,
        grid=(N // BLK,),
        in_specs=[
            pl.BlockSpec((BLK, D), lambda i: (i, 0)),
            pl.BlockSpec((BLK, D), lambda i: (i, 0)),
            pl.BlockSpec((BLK, 1), lambda i: (i, 0)),
            full((H1, D)), full((H1,)),
            full((H2, H1)), full((H2,)),
            full((H3, H2)), full((H3,)),
            full((1, H3)), pl.BlockSpec(memory_space=pltpu.SMEM),
        ],
        out_specs=pl.BlockSpec((BLK, 1), lambda i: (i, 0)),
        out_shape=jax.ShapeDtypeStruct((N, 1), jnp.float32),
    )(xs, acc_p, cinv, W_conv, b_conv, W1, b1, W2, b2, W3, b3.reshape(1, 1))


def kernel(x, edge_index, W_conv, b_conv, W1, b1, W2, b2, W3, b3):
    packed = (edge_index[0] << SHIFT) | edge_index[1]
    deg_p, cnt_p = _k1(packed.reshape(NW, NB1, CB1, CH))
    xs, cinv = _k2(x, deg_p.reshape(NC, NPAD, 1), cnt_p.reshape(NC, NPAD, 1))
    acc_p = _k3(packed.reshape(NS, NB2, CB2, CH), xs)
    return _k4(xs, acc_p, cinv, W_conv, b_conv, W1, b1, W2, b2, W3, b3)
